# speedup vs baseline: 52.6956x; 52.6956x over previous
import jax
import jax.numpy as jnp
from jax.experimental import pallas as pl
from jax.experimental.pallas import tpu as pltpu

B, C, H, W_ = 8, 96, 32, 32
N = H * W_
K = 9
G = 4
CG = C // G
EPS_GN = 1e-5
NEG = -3.0e38


def _gcn_block_kernel(x_ref, w1_ref, b1_ref, w2_ref, b2_ref,
                      g1w_ref, g1b_ref, g2w_ref, g2b_ref, out_ref):
    xf = x_ref[0]

    nrm = jnp.sqrt(jnp.sum(xf * xf, axis=1, keepdims=True))
    xn = xf / jnp.maximum(nrm, 1e-12)

    sim = jax.lax.dot_general(
        xn, xn, dimension_numbers=(((1,), (1,)), ((), ())),
        preferred_element_type=jnp.float32)

    cur = sim
    for _ in range(K - 1):
        m = jnp.max(cur, axis=1, keepdims=True)
        cur = jnp.where(cur >= m, NEG, cur)
    thresh = jnp.max(cur, axis=1, keepdims=True)

    w = jnp.where(sim >= thresh, sim, 0.0)
    deg = jnp.sum(w, axis=1, keepdims=True) + 1e-6
    adj = w / deg

    c_io = jax.lax.broadcasted_iota(jnp.int32, (G, C), 1)
    g_io = jax.lax.broadcasted_iota(jnp.int32, (G, C), 0)
    mt = (c_io // CG == g_io).astype(jnp.float32)

    def gcn_gn_silu(h_in, w_ref, b_ref, gw_ref, gb_ref):
        xt = jnp.dot(h_in, w_ref[...], preferred_element_type=jnp.float32)
        h = jnp.dot(adj, xt, preferred_element_type=jnp.float32) + b_ref[...]
        s = jnp.sum(h, axis=0, keepdims=True)
        ss = jnp.sum(h * h, axis=0, keepdims=True)
        gs = jax.lax.dot_general(
            s, mt, dimension_numbers=(((1,), (1,)), ((), ())),
            preferred_element_type=jnp.float32)
        gss = jax.lax.dot_general(
            ss, mt, dimension_numbers=(((1,), (1,)), ((), ())),
            preferred_element_type=jnp.float32)
        cnt = float(N * CG)
        mean_g = gs / cnt
        var_g = gss / cnt - mean_g * mean_g
        mean_c = jnp.dot(mean_g, mt, preferred_element_type=jnp.float32)
        var_c = jnp.dot(var_g, mt, preferred_element_type=jnp.float32)
        inv = jax.lax.rsqrt(var_c + EPS_GN)
        hn = (h - mean_c) * inv * gw_ref[...] + gb_ref[...]
        return hn * jax.nn.sigmoid(hn)

    s1 = gcn_gn_silu(xn, w1_ref, b1_ref, g1w_ref, g1b_ref)
    s2 = gcn_gn_silu(s1, w2_ref, b2_ref, g2w_ref, g2b_ref)
    out_ref[0] = s2


def kernel(x, W1, b1, W2, b2, gn1_w, gn1_b, gn2_w, gn2_b):
    xf = x.reshape(B, C, N).transpose(0, 2, 1)
    vec = lambda v: v.reshape(1, C)
    full = lambda shape: pl.BlockSpec(shape, lambda b: (0,) * len(shape))

    y = pl.pallas_call(
        _gcn_block_kernel,
        grid=(B,),
        in_specs=[
            pl.BlockSpec((1, N, C), lambda b: (b, 0, 0)),
            full((C, C)), full((1, C)),
            full((C, C)), full((1, C)),
            full((1, C)), full((1, C)),
            full((1, C)), full((1, C)),
        ],
        out_specs=pl.BlockSpec((1, N, C), lambda b: (b, 0, 0)),
        out_shape=jax.ShapeDtypeStruct((B, N, C), jnp.float32),
        compiler_params=pltpu.CompilerParams(
            dimension_semantics=("arbitrary",)),
    )(xf, W1, vec(b1), W2, vec(b2),
      vec(gn1_w), vec(gn1_b), vec(gn2_w), vec(gn2_b))

    return y.transpose(0, 2, 1).reshape(B, C, H, W_)


# speedup vs baseline: 56.0353x; 1.0634x over previous
import jax
import jax.numpy as jnp
from jax.experimental import pallas as pl
from jax.experimental.pallas import tpu as pltpu

B, C, H, W_ = 8, 96, 32, 32
N = H * W_
K = 9
G = 4
CG = C // G
EPS_GN = 1e-5
NEG = -3.0e38


def _gcn_block_kernel(x_ref, w1_ref, b1_ref, w2_ref, b2_ref,
                      g1w_ref, g1b_ref, g2w_ref, g2b_ref, out_ref):
    xf = x_ref[0]

    nrm = jnp.sqrt(jnp.sum(xf * xf, axis=1, keepdims=True))
    xn = xf / jnp.maximum(nrm, 1e-12)

    sim = jax.lax.dot_general(
        xn, xn, dimension_numbers=(((1,), (1,)), ((), ())),
        preferred_element_type=jnp.float32)

    m = jnp.max(sim, axis=1, keepdims=True)
    deg = m
    for _ in range(K - 1):
        m = jnp.max(jnp.where(sim < m, sim, NEG), axis=1, keepdims=True)
        deg = deg + m
    thresh = m
    deg = deg + 1e-6

    w = jnp.where(sim >= thresh, sim, 0.0)

    c_io = jax.lax.broadcasted_iota(jnp.int32, (G, C), 1)
    g_io = jax.lax.broadcasted_iota(jnp.int32, (G, C), 0)
    mt = (c_io // CG == g_io).astype(jnp.float32)

    def gcn_gn_silu(h_in, w_ref, b_ref, gw_ref, gb_ref):
        xt = jnp.dot(h_in, w_ref[...], preferred_element_type=jnp.float32)
        h = (jnp.dot(w, xt, preferred_element_type=jnp.float32) / deg
             + b_ref[...])
        s = jnp.sum(h, axis=0, keepdims=True)
        ss = jnp.sum(h * h, axis=0, keepdims=True)
        gs = jax.lax.dot_general(
            s, mt, dimension_numbers=(((1,), (1,)), ((), ())),
            preferred_element_type=jnp.float32)
        gss = jax.lax.dot_general(
            ss, mt, dimension_numbers=(((1,), (1,)), ((), ())),
            preferred_element_type=jnp.float32)
        cnt = float(N * CG)
        mean_g = gs / cnt
        var_g = gss / cnt - mean_g * mean_g
        mean_c = jnp.dot(mean_g, mt, preferred_element_type=jnp.float32)
        var_c = jnp.dot(var_g, mt, preferred_element_type=jnp.float32)
        inv = jax.lax.rsqrt(var_c + EPS_GN)
        hn = (h - mean_c) * inv * gw_ref[...] + gb_ref[...]
        return hn * jax.nn.sigmoid(hn)

    s1 = gcn_gn_silu(xn, w1_ref, b1_ref, g1w_ref, g1b_ref)
    s2 = gcn_gn_silu(s1, w2_ref, b2_ref, g2w_ref, g2b_ref)
    out_ref[0] = s2


def kernel(x, W1, b1, W2, b2, gn1_w, gn1_b, gn2_w, gn2_b):
    xf = x.reshape(B, C, N).transpose(0, 2, 1)
    vec = lambda v: v.reshape(1, C)
    full = lambda shape: pl.BlockSpec(shape, lambda b: (0,) * len(shape))

    y = pl.pallas_call(
        _gcn_block_kernel,
        grid=(B,),
        in_specs=[
            pl.BlockSpec((1, N, C), lambda b: (b, 0, 0)),
            full((C, C)), full((1, C)),
            full((C, C)), full((1, C)),
            full((1, C)), full((1, C)),
            full((1, C)), full((1, C)),
        ],
        out_specs=pl.BlockSpec((1, N, C), lambda b: (b, 0, 0)),
        out_shape=jax.ShapeDtypeStruct((B, N, C), jnp.float32),
        compiler_params=pltpu.CompilerParams(
            dimension_semantics=("arbitrary",)),
    )(xf, W1, vec(b1), W2, vec(b2),
      vec(gn1_w), vec(gn1_b), vec(gn2_w), vec(gn2_b))

    return y.transpose(0, 2, 1).reshape(B, C, H, W_)
